# Initial kernel scaffold; baseline (speedup 1.0000x reference)
import jax
import jax.numpy as jnp
from jax.experimental import pallas as pl


def kernel(x, edge_index, edge_attr, Wx, bx, We0, be0, W1_0, b1_0, W2_0, b2_0, gamma0, beta0, We1, be1, W1_1, b1_1, W2_1, b2_1, gamma1, beta1):
    raise NotImplementedError("write your pallas kernel here")


# speedup vs baseline: 3.0612x; 3.0612x over previous
import functools

import jax
import jax.numpy as jnp
from jax import lax
from jax.experimental import pallas as pl
from jax.experimental.pallas import tpu as pltpu
from jax.experimental.pallas import tpu_sc as plsc

_N = 10000
_E = 320000
_EMB = 128

_NC = 2
_NS = 16
_NW = _NC * _NS
_CH = 128
_EW = 10240
_EP = _NW * _EW
_R = 10240
_RT = _R // _NS
_DUMMY = _N
_EAW = 16

_BLK = 1000

_SC_DO_LOOP = True


def _sc_body(gather, width, *refs):
    if gather:
        (vals_hbm, src_hbm, dst_hbm, z_hbm, out_hbm,
         acc_sh, src_v, dst_v, rows_v, sem) = refs
    else:
        (vals_hbm, dst_hbm, z_hbm, out_hbm,
         acc_sh, dst_v, rows_v, flat_v, sem) = refs
    cid = lax.axis_index("c")
    sid = lax.axis_index("s")
    w = cid * _NS + sid
    r0 = sid * _RT
    obase = cid * _R + r0

    pltpu.sync_copy(z_hbm, rows_v)
    for j in range(_RT // _CH):
        pltpu.sync_copy(rows_v, acc_sh.at[pl.ds(r0 + j * _CH, _CH)])
    plsc.subcore_barrier()

    def chunk(i, carry):
        base = pl.multiple_of(w * _EW + i * _CH, _CH)
        pltpu.sync_copy(dst_hbm.at[pl.ds(base, _CH)], dst_v)
        if gather:
            pltpu.sync_copy(src_hbm.at[pl.ds(base, _CH)], src_v)
            pltpu.async_copy(vals_hbm.at[src_v], rows_v, sem).wait()
        else:
            pltpu.sync_copy(
                vals_hbm.at[pl.ds(base * _EAW, _CH * _EAW)], flat_v)
            for k in range(_CH):
                rows_v[k, pl.ds(0, _EAW)] = flat_v[pl.ds(k * _EAW, _EAW)]
        pltpu.sync_copy(rows_v, acc_sh.at[dst_v], add=True)
        return carry

    lax.fori_loop(0, _EW // _CH, chunk, 0)
    plsc.subcore_barrier()

    for j in range(_RT // _CH):
        pltpu.sync_copy(acc_sh.at[pl.ds(r0 + j * _CH, _CH)], rows_v)
        pltpu.sync_copy(rows_v, out_hbm.at[pl.ds(obase + j * _CH, _CH)])


def _make_sc(gather, width):
    mesh = plsc.VectorSubcoreMesh(
        core_axis_name="c", subcore_axis_name="s",
        num_cores=_NC, num_subcores=_NS)
    scratch = [
        pltpu.VMEM_SHARED((_R, width), jnp.float32),
        pltpu.VMEM((_CH,), jnp.int32),
        pltpu.VMEM((_CH, width), jnp.float32),
        pltpu.SemaphoreType.DMA,
    ]
    out_type = jax.ShapeDtypeStruct((_NC * _R, width), jnp.float32)
    if gather:
        scratch.insert(1, pltpu.VMEM((_CH,), jnp.int32))
    else:
        scratch.insert(3, pltpu.VMEM((_CH * _EAW,), jnp.float32))
    return pl.kernel(
        functools.partial(_sc_body, gather, width),
        out_type=out_type,
        mesh=mesh,
        scratch_types=scratch,
    )


_sc_seg_h = _make_sc(True, _EMB)
_sc_seg_ea = _make_sc(False, _EMB)


def _h0_body(x_ref, w_ref, b_ref, o_ref):
    o_ref[...] = (jnp.dot(x_ref[...], w_ref[...],
                          preferred_element_type=jnp.float32) + b_ref[...])


_tc_h0 = pl.pallas_call(
    _h0_body,
    grid=(_N // _BLK,),
    in_specs=[
        pl.BlockSpec((_BLK, 8), lambda i: (i, 0)),
        pl.BlockSpec((8, _EMB), lambda i: (0, 0)),
        pl.BlockSpec((1, _EMB), lambda i: (0, 0)),
    ],
    out_specs=pl.BlockSpec((_BLK, _EMB), lambda i: (i, 0)),
    out_shape=jax.ShapeDtypeStruct((_N, _EMB), jnp.float32),
)


def _mlp_body(p_ref, h_ref, a_ref, we_ref, c_ref, w1_ref, b1_ref, w2_ref,
              b2_ref, g_ref, s_ref, ss_ref):
    i = pl.program_id(0)
    agg = a_ref[0] + a_ref[1]
    aggr = (p_ref[0] + p_ref[1] + h_ref[...] + c_ref[...]
            + jnp.dot(agg, we_ref[...], preferred_element_type=jnp.float32))
    hid = jnp.maximum(
        jnp.dot(aggr, w1_ref[...], preferred_element_type=jnp.float32)
        + b1_ref[...], 0.0)
    g = (jnp.dot(hid, w2_ref[...], preferred_element_type=jnp.float32)
         + b2_ref[...])
    g_ref[...] = g

    @pl.when(i == 0)
    def _():
        s_ref[...] = jnp.zeros_like(s_ref)
        ss_ref[...] = jnp.zeros_like(ss_ref)

    s_ref[...] += jnp.sum(g, axis=0, keepdims=True)
    ss_ref[...] += jnp.sum(g * g, axis=0, keepdims=True)


_tc_mlp = pl.pallas_call(
    _mlp_body,
    grid=(_N // _BLK,),
    in_specs=[
        pl.BlockSpec((2, _BLK, _EMB), lambda i: (0, i, 0)),
        pl.BlockSpec((_BLK, _EMB), lambda i: (i, 0)),
        pl.BlockSpec((2, _BLK, _EMB), lambda i: (0, i, 0)),
        pl.BlockSpec((_EMB, _EMB), lambda i: (0, 0)),
        pl.BlockSpec((1, _EMB), lambda i: (0, 0)),
        pl.BlockSpec((_EMB, 2 * _EMB), lambda i: (0, 0)),
        pl.BlockSpec((1, 2 * _EMB), lambda i: (0, 0)),
        pl.BlockSpec((2 * _EMB, _EMB), lambda i: (0, 0)),
        pl.BlockSpec((1, _EMB), lambda i: (0, 0)),
    ],
    out_specs=[
        pl.BlockSpec((_BLK, _EMB), lambda i: (i, 0)),
        pl.BlockSpec((1, _EMB), lambda i: (0, 0)),
        pl.BlockSpec((1, _EMB), lambda i: (0, 0)),
    ],
    out_shape=[
        jax.ShapeDtypeStruct((_N, _EMB), jnp.float32),
        jax.ShapeDtypeStruct((1, _EMB), jnp.float32),
        jax.ShapeDtypeStruct((1, _EMB), jnp.float32),
    ],
)


def _bn_body(do_elu, g_ref, s_ref, ss_ref, gam_ref, bet_ref, o_ref):
    m = s_ref[...] * (1.0 / _N)
    v = ss_ref[...] * (1.0 / _N) - m * m
    y = gam_ref[...] * (g_ref[...] - m) * lax.rsqrt(v + 1e-5) + bet_ref[...]
    if do_elu:
        y = jnp.where(y > 0, y, jnp.exp(y) - 1.0)
    o_ref[...] = y


def _make_bn(do_elu):
    return pl.pallas_call(
        functools.partial(_bn_body, do_elu),
        grid=(_N // _BLK,),
        in_specs=[
            pl.BlockSpec((_BLK, _EMB), lambda i: (i, 0)),
            pl.BlockSpec((1, _EMB), lambda i: (0, 0)),
            pl.BlockSpec((1, _EMB), lambda i: (0, 0)),
            pl.BlockSpec((1, _EMB), lambda i: (0, 0)),
            pl.BlockSpec((1, _EMB), lambda i: (0, 0)),
        ],
        out_specs=pl.BlockSpec((_BLK, _EMB), lambda i: (i, 0)),
        out_shape=jax.ShapeDtypeStruct((_N, _EMB), jnp.float32),
    )


_tc_bn_elu = _make_bn(True)
_tc_bn = _make_bn(False)


def kernel(x, edge_index, edge_attr, Wx, bx, We0, be0, W1_0, b1_0, W2_0,
           b2_0, gamma0, beta0, We1, be1, W1_1, b1_1, W2_1, b2_1, gamma1,
           beta1):
    f32 = jnp.float32
    pad = _EP - _E

    srcp = jnp.concatenate([edge_index[0], jnp.zeros((pad,), jnp.int32)])
    dstp = jnp.concatenate(
        [edge_index[1], jnp.full((pad,), _DUMMY, jnp.int32)])
    ea16 = jnp.concatenate(
        [edge_attr, jnp.ones((_E, 1), f32), jnp.zeros((_E, _EAW - 6), f32)],
        axis=1)
    ea16 = jnp.concatenate([ea16, jnp.zeros((pad, _EAW), f32)], axis=0)
    zr = jnp.zeros((_CH, _EMB), f32)

    x8 = jnp.concatenate([x, jnp.zeros((_N, 1), f32)], axis=1)
    Wx8 = jnp.concatenate([Wx, jnp.zeros((1, _EMB), f32)], axis=0)
    h0 = _tc_h0(x8, Wx8, bx.reshape(1, -1))

    P = _sc_seg_h(h0, srcp, dstp, zr)
    A = _sc_seg_ea(ea16.reshape(_EP * _EAW), dstp, zr)
    Pn = P.reshape(_NC, _R, _EMB)[:, :_N, :]
    An = A.reshape(_NC, _R, _EMB)[:, :_N, :]

    def dense_layer(h, Pn, An, We, be, W1, b1, W2, b2, gamma, beta, bn):
        Wem = jnp.concatenate(
            [We, be.reshape(1, -1), jnp.zeros((_EMB - 6, _EMB), f32)], axis=0)
        c = (8.0 * We[0] + be).reshape(1, -1)
        g, s, ss = _tc_mlp(Pn, h, An, Wem, c, W1, b1.reshape(1, -1), W2,
                           b2.reshape(1, -1))
        return bn(g, s, ss, gamma.reshape(1, -1), beta.reshape(1, -1))

    h1 = dense_layer(h0, Pn, An, We0, be0, W1_0, b1_0, W2_0, b2_0,
                     gamma0, beta0, _tc_bn_elu)

    P1 = _sc_seg_h(h1, srcp, dstp, zr).reshape(_NC, _R, _EMB)
    h2 = dense_layer(h1, P1[:, :_N, :], An, We1, be1, W1_1, b1_1, W2_1,
                     b2_1, gamma1, beta1, _tc_bn)
    return h2


# speedup vs baseline: 3.4519x; 1.1276x over previous
import functools

import jax
import jax.numpy as jnp
from jax import lax
from jax.experimental import pallas as pl
from jax.experimental.pallas import tpu as pltpu
from jax.experimental.pallas import tpu_sc as plsc

_N = 10000
_E = 320000
_EMB = 128

_NC = 2
_NS = 16
_NW = _NC * _NS
_CH = 128
_EW = 10240
_EP = _NW * _EW
_R = 10240
_RT = _R // _NS
_DUMMY = _N
_EAW = 16

_BLK = 1000


def _sc_common(refs_sid):
    cid = lax.axis_index("c")
    sid = lax.axis_index("s")
    w = cid * _NS + sid
    r0 = sid * _RT
    obase = cid * _R + r0
    return cid, sid, w, r0, obase


def _zero_acc(z_hbm, rows_v, acc_sh, r0):
    pltpu.sync_copy(z_hbm, rows_v)
    for j in range(_RT // _CH):
        pltpu.sync_copy(rows_v, acc_sh.at[pl.ds(r0 + j * _CH, _CH)])


def _copy_out(acc_sh, rows_v, out_hbm, r0, obase):
    for j in range(_RT // _CH):
        pltpu.sync_copy(acc_sh.at[pl.ds(r0 + j * _CH, _CH)], rows_v)
        pltpu.sync_copy(rows_v, out_hbm.at[pl.ds(obase + j * _CH, _CH)])


def _sc_gather_body(vals_hbm, idx_hbm, z_hbm, out_hbm, acc_sh, idx_db,
                    rows_a, rows_b, sem_ia, sem_ib, sem_ga, sem_gb):
    _, sid, w, r0, obase = _sc_common(None)
    nck = _EW // _CH

    _zero_acc(z_hbm, rows_a, acc_sh, r0)
    plsc.subcore_barrier()

    def idx_issue(i, lane, sem):
        pltpu.async_copy(idx_hbm.at[w * nck + i], idx_db.at[lane], sem)

    def idx_wait(i, lane, sem):
        pltpu.make_async_copy(
            idx_hbm.at[w * nck + i], idx_db.at[lane], sem).wait()

    def g_issue(lane, rows_v, sem):
        pltpu.async_copy(vals_hbm.at[idx_db.at[lane, 0]], rows_v, sem)

    def g_wait(lane, rows_v, sem):
        pltpu.make_async_copy(
            vals_hbm.at[idx_db.at[lane, 0]], rows_v, sem).wait()

    idx_issue(0, 0, sem_ia)
    idx_issue(1, 1, sem_ib)
    idx_wait(0, 0, sem_ia)
    g_issue(0, rows_a, sem_ga)

    def pair(g, carry):
        i0 = 2 * g
        i1 = i0 + 1
        idx_wait(i1, 1, sem_ib)
        g_wait(0, rows_a, sem_ga)
        g_issue(1, rows_b, sem_gb)
        pltpu.sync_copy(rows_a, acc_sh.at[idx_db.at[0, 1]], add=True)

        @pl.when(i0 + 2 < nck)
        def _():
            idx_issue(i0 + 2, 0, sem_ia)

        g_wait(1, rows_b, sem_gb)

        @pl.when(i0 + 2 < nck)
        def _():
            idx_wait(i0 + 2, 0, sem_ia)
            g_issue(0, rows_a, sem_ga)

        pltpu.sync_copy(rows_b, acc_sh.at[idx_db.at[1, 1]], add=True)

        @pl.when(i1 + 2 < nck)
        def _():
            idx_issue(i1 + 2, 1, sem_ib)

        return carry

    lax.fori_loop(0, nck // 2, pair, 0)
    plsc.subcore_barrier()
    _copy_out(acc_sh, rows_a, out_hbm, r0, obase)


def _sc_ea_body(vals_hbm, idx_hbm, z_hbm, out_hbm, acc_sh, idx_db,
                rows_a, flat_a, flat_b, sem_ia, sem_ib, sem_fa, sem_fb):
    _, sid, w, r0, obase = _sc_common(None)
    nck = _EW // _CH

    _zero_acc(z_hbm, rows_a, acc_sh, r0)
    plsc.subcore_barrier()

    def idx_issue(i, lane, sem):
        pltpu.async_copy(idx_hbm.at[w * nck + i], idx_db.at[lane], sem)

    def idx_wait(i, lane, sem):
        pltpu.make_async_copy(
            idx_hbm.at[w * nck + i], idx_db.at[lane], sem).wait()

    def f_issue(i, flat_v, sem):
        pltpu.async_copy(
            vals_hbm.at[pl.ds((w * nck + i) * _CH * _EAW, _CH * _EAW)],
            flat_v, sem)

    def f_wait(i, flat_v, sem):
        pltpu.make_async_copy(
            vals_hbm.at[pl.ds((w * nck + i) * _CH * _EAW, _CH * _EAW)],
            flat_v, sem).wait()

    idx_issue(0, 0, sem_ia)
    idx_issue(1, 1, sem_ib)
    f_issue(0, flat_a, sem_fa)
    f_issue(1, flat_b, sem_fb)

    def step(i, lane, flat_v, sem_i, sem_f):
        idx_wait(i, lane, sem_i)
        f_wait(i, flat_v, sem_f)
        for k in range(_CH):
            rows_a[k, pl.ds(0, _EAW)] = flat_v[pl.ds(k * _EAW, _EAW)]
        pltpu.sync_copy(rows_a, acc_sh.at[idx_db.at[lane, 1]], add=True)

        @pl.when(i + 2 < nck)
        def _():
            idx_issue(i + 2, lane, sem_i)
            f_issue(i + 2, flat_v, sem_f)

    def pair(g, carry):
        step(2 * g, 0, flat_a, sem_ia, sem_fa)
        step(2 * g + 1, 1, flat_b, sem_ib, sem_fb)
        return carry

    lax.fori_loop(0, nck // 2, pair, 0)
    plsc.subcore_barrier()
    _copy_out(acc_sh, rows_a, out_hbm, r0, obase)


def _make_sc(gather, width):
    mesh = plsc.VectorSubcoreMesh(
        core_axis_name="c", subcore_axis_name="s",
        num_cores=_NC, num_subcores=_NS)
    if gather:
        body = _sc_gather_body
        scratch = [
            pltpu.VMEM_SHARED((_R, width), jnp.float32),
            pltpu.VMEM((2, 2, _CH), jnp.int32),
            pltpu.VMEM((_CH, width), jnp.float32),
            pltpu.VMEM((_CH, width), jnp.float32),
            pltpu.SemaphoreType.DMA,
            pltpu.SemaphoreType.DMA,
            pltpu.SemaphoreType.DMA,
            pltpu.SemaphoreType.DMA,
        ]
    else:
        body = _sc_ea_body
        scratch = [
            pltpu.VMEM_SHARED((_R, width), jnp.float32),
            pltpu.VMEM((2, 2, _CH), jnp.int32),
            pltpu.VMEM((_CH, width), jnp.float32),
            pltpu.VMEM((_CH * _EAW,), jnp.float32),
            pltpu.VMEM((_CH * _EAW,), jnp.float32),
            pltpu.SemaphoreType.DMA,
            pltpu.SemaphoreType.DMA,
            pltpu.SemaphoreType.DMA,
            pltpu.SemaphoreType.DMA,
        ]
    out_type = jax.ShapeDtypeStruct((_NC * _R, width), jnp.float32)
    return pl.kernel(
        body,
        out_type=out_type,
        mesh=mesh,
        scratch_types=scratch,
    )


_sc_seg_h = _make_sc(True, _EMB)
_sc_seg_ea = _make_sc(False, _EMB)


def _h0_body(x_ref, w_ref, b_ref, o_ref):
    o_ref[...] = (jnp.dot(x_ref[...], w_ref[...],
                          preferred_element_type=jnp.float32) + b_ref[...])


_tc_h0 = pl.pallas_call(
    _h0_body,
    grid=(_N // _BLK,),
    in_specs=[
        pl.BlockSpec((_BLK, 8), lambda i: (i, 0)),
        pl.BlockSpec((8, _EMB), lambda i: (0, 0)),
        pl.BlockSpec((1, _EMB), lambda i: (0, 0)),
    ],
    out_specs=pl.BlockSpec((_BLK, _EMB), lambda i: (i, 0)),
    out_shape=jax.ShapeDtypeStruct((_N, _EMB), jnp.float32),
)


def _mlp_body(p_ref, h_ref, a_ref, we_ref, c_ref, w1_ref, b1_ref, w2_ref,
              b2_ref, g_ref, s_ref, ss_ref):
    i = pl.program_id(0)
    agg = a_ref[0] + a_ref[1]
    aggr = (p_ref[0] + p_ref[1] + h_ref[...] + c_ref[...]
            + jnp.dot(agg, we_ref[...], preferred_element_type=jnp.float32))
    hid = jnp.maximum(
        jnp.dot(aggr, w1_ref[...], preferred_element_type=jnp.float32)
        + b1_ref[...], 0.0)
    g = (jnp.dot(hid, w2_ref[...], preferred_element_type=jnp.float32)
         + b2_ref[...])
    g_ref[...] = g

    @pl.when(i == 0)
    def _():
        s_ref[...] = jnp.zeros_like(s_ref)
        ss_ref[...] = jnp.zeros_like(ss_ref)

    s_ref[...] += jnp.sum(g, axis=0, keepdims=True)
    ss_ref[...] += jnp.sum(g * g, axis=0, keepdims=True)


_tc_mlp = pl.pallas_call(
    _mlp_body,
    grid=(_N // _BLK,),
    in_specs=[
        pl.BlockSpec((2, _BLK, _EMB), lambda i: (0, i, 0)),
        pl.BlockSpec((_BLK, _EMB), lambda i: (i, 0)),
        pl.BlockSpec((2, _BLK, _EMB), lambda i: (0, i, 0)),
        pl.BlockSpec((_EMB, _EMB), lambda i: (0, 0)),
        pl.BlockSpec((1, _EMB), lambda i: (0, 0)),
        pl.BlockSpec((_EMB, 2 * _EMB), lambda i: (0, 0)),
        pl.BlockSpec((1, 2 * _EMB), lambda i: (0, 0)),
        pl.BlockSpec((2 * _EMB, _EMB), lambda i: (0, 0)),
        pl.BlockSpec((1, _EMB), lambda i: (0, 0)),
    ],
    out_specs=[
        pl.BlockSpec((_BLK, _EMB), lambda i: (i, 0)),
        pl.BlockSpec((1, _EMB), lambda i: (0, 0)),
        pl.BlockSpec((1, _EMB), lambda i: (0, 0)),
    ],
    out_shape=[
        jax.ShapeDtypeStruct((_N, _EMB), jnp.float32),
        jax.ShapeDtypeStruct((1, _EMB), jnp.float32),
        jax.ShapeDtypeStruct((1, _EMB), jnp.float32),
    ],
)


def _bn_body(do_elu, g_ref, s_ref, ss_ref, gam_ref, bet_ref, o_ref):
    m = s_ref[...] * (1.0 / _N)
    v = ss_ref[...] * (1.0 / _N) - m * m
    y = gam_ref[...] * (g_ref[...] - m) * lax.rsqrt(v + 1e-5) + bet_ref[...]
    if do_elu:
        y = jnp.where(y > 0, y, jnp.exp(y) - 1.0)
    o_ref[...] = y


def _make_bn(do_elu):
    return pl.pallas_call(
        functools.partial(_bn_body, do_elu),
        grid=(_N // _BLK,),
        in_specs=[
            pl.BlockSpec((_BLK, _EMB), lambda i: (i, 0)),
            pl.BlockSpec((1, _EMB), lambda i: (0, 0)),
            pl.BlockSpec((1, _EMB), lambda i: (0, 0)),
            pl.BlockSpec((1, _EMB), lambda i: (0, 0)),
            pl.BlockSpec((1, _EMB), lambda i: (0, 0)),
        ],
        out_specs=pl.BlockSpec((_BLK, _EMB), lambda i: (i, 0)),
        out_shape=jax.ShapeDtypeStruct((_N, _EMB), jnp.float32),
    )


_tc_bn_elu = _make_bn(True)
_tc_bn = _make_bn(False)


def kernel(x, edge_index, edge_attr, Wx, bx, We0, be0, W1_0, b1_0, W2_0,
           b2_0, gamma0, beta0, We1, be1, W1_1, b1_1, W2_1, b2_1, gamma1,
           beta1):
    f32 = jnp.float32
    pad = _EP - _E

    srcp = jnp.concatenate([edge_index[0], jnp.zeros((pad,), jnp.int32)])
    dstp = jnp.concatenate(
        [edge_index[1], jnp.full((pad,), _DUMMY, jnp.int32)])
    ea16 = jnp.concatenate(
        [edge_attr, jnp.ones((_E, 1), f32), jnp.zeros((_E, _EAW - 6), f32)],
        axis=1)
    ea16 = jnp.concatenate([ea16, jnp.zeros((pad, _EAW), f32)], axis=0)
    zr = jnp.zeros((_CH, _EMB), f32)

    x8 = jnp.concatenate([x, jnp.zeros((_N, 1), f32)], axis=1)
    Wx8 = jnp.concatenate([Wx, jnp.zeros((1, _EMB), f32)], axis=0)
    h0 = _tc_h0(x8, Wx8, bx.reshape(1, -1))

    ei3 = jnp.stack([srcp.reshape(_EP // _CH, _CH),
                     dstp.reshape(_EP // _CH, _CH)], axis=1)
    P = _sc_seg_h(h0, ei3, zr)
    A = _sc_seg_ea(ea16.reshape(_EP * _EAW), ei3, zr)
    Pn = P.reshape(_NC, _R, _EMB)[:, :_N, :]
    An = A.reshape(_NC, _R, _EMB)[:, :_N, :]

    def dense_layer(h, Pn, An, We, be, W1, b1, W2, b2, gamma, beta, bn):
        Wem = jnp.concatenate(
            [We, be.reshape(1, -1), jnp.zeros((_EMB - 6, _EMB), f32)], axis=0)
        c = (8.0 * We[0] + be).reshape(1, -1)
        g, s, ss = _tc_mlp(Pn, h, An, Wem, c, W1, b1.reshape(1, -1), W2,
                           b2.reshape(1, -1))
        return bn(g, s, ss, gamma.reshape(1, -1), beta.reshape(1, -1))

    h1 = dense_layer(h0, Pn, An, We0, be0, W1_0, b1_0, W2_0, b2_0,
                     gamma0, beta0, _tc_bn_elu)

    P1 = _sc_seg_h(h1, ei3, zr).reshape(_NC, _R, _EMB)
    h2 = dense_layer(h1, P1[:, :_N, :], An, We1, be1, W1_1, b1_1, W2_1,
                     b2_1, gamma1, beta1, _tc_bn)
    return h2


# speedup vs baseline: 3.4529x; 1.0003x over previous
import functools

import jax
import jax.numpy as jnp
from jax import lax
from jax.experimental import pallas as pl
from jax.experimental.pallas import tpu as pltpu
from jax.experimental.pallas import tpu_sc as plsc

_N = 10000
_E = 320000
_EMB = 128

_NC = 2
_NS = 16
_NW = _NC * _NS
_CH = 128
_EW = 10240
_EP = _NW * _EW
_R = 10240
_RT = _R // _NS
_DUMMY = _N
_EAW = 16

_BLK = 1000


def _sc_common(refs_sid):
    cid = lax.axis_index("c")
    sid = lax.axis_index("s")
    w = cid * _NS + sid
    r0 = sid * _RT
    obase = cid * _R + r0
    return cid, sid, w, r0, obase


def _zero_acc(z_hbm, rows_v, acc_sh, r0):
    pltpu.sync_copy(z_hbm, rows_v)
    for j in range(_RT // _CH):
        pltpu.sync_copy(rows_v, acc_sh.at[pl.ds(r0 + j * _CH, _CH)])


def _copy_out(acc_sh, rows_v, out_hbm, r0, obase):
    for j in range(_RT // _CH):
        pltpu.sync_copy(acc_sh.at[pl.ds(r0 + j * _CH, _CH)], rows_v)
        pltpu.sync_copy(rows_v, out_hbm.at[pl.ds(obase + j * _CH, _CH)])


def _sc_gather_body(vals_hbm, idx_hbm, z_hbm, out_hbm, acc_sh, idx_db,
                    rows_a, rows_b, sem_ia, sem_ib, sem_ga, sem_gb):
    _, sid, w, r0, obase = _sc_common(None)
    nck = _EW // _CH

    _zero_acc(z_hbm, rows_a, acc_sh, r0)
    plsc.subcore_barrier()

    def idx_issue(i, lane, sem):
        pltpu.async_copy(idx_hbm.at[w * nck + i], idx_db.at[lane], sem)

    def idx_wait(i, lane, sem):
        pltpu.make_async_copy(
            idx_hbm.at[w * nck + i], idx_db.at[lane], sem).wait()

    nsp = 4
    part = _CH // nsp

    def g_issue(lane, rows_v, sem):
        for p in range(nsp):
            pltpu.async_copy(
                vals_hbm.at[idx_db.at[lane, 0, pl.ds(p * part, part)]],
                rows_v.at[pl.ds(p * part, part)], sem)

    def g_wait(lane, rows_v, sem):
        for p in range(nsp):
            pltpu.make_async_copy(
                vals_hbm.at[idx_db.at[lane, 0, pl.ds(p * part, part)]],
                rows_v.at[pl.ds(p * part, part)], sem).wait()

    idx_issue(0, 0, sem_ia)
    idx_issue(1, 1, sem_ib)
    idx_wait(0, 0, sem_ia)
    g_issue(0, rows_a, sem_ga)

    def pair(g, carry):
        i0 = 2 * g
        i1 = i0 + 1
        idx_wait(i1, 1, sem_ib)
        g_wait(0, rows_a, sem_ga)
        g_issue(1, rows_b, sem_gb)
        pltpu.sync_copy(rows_a, acc_sh.at[idx_db.at[0, 1]], add=True)

        @pl.when(i0 + 2 < nck)
        def _():
            idx_issue(i0 + 2, 0, sem_ia)

        g_wait(1, rows_b, sem_gb)

        @pl.when(i0 + 2 < nck)
        def _():
            idx_wait(i0 + 2, 0, sem_ia)
            g_issue(0, rows_a, sem_ga)

        pltpu.sync_copy(rows_b, acc_sh.at[idx_db.at[1, 1]], add=True)

        @pl.when(i1 + 2 < nck)
        def _():
            idx_issue(i1 + 2, 1, sem_ib)

        return carry

    lax.fori_loop(0, nck // 2, pair, 0)
    plsc.subcore_barrier()
    _copy_out(acc_sh, rows_a, out_hbm, r0, obase)


def _sc_ea_body(vals_hbm, idx_hbm, z_hbm, out_hbm, acc_sh, idx_db,
                rows_a, flat_a, flat_b, sem_ia, sem_ib, sem_fa, sem_fb):
    _, sid, w, r0, obase = _sc_common(None)
    nck = _EW // _CH

    _zero_acc(z_hbm, rows_a, acc_sh, r0)
    plsc.subcore_barrier()

    def idx_issue(i, lane, sem):
        pltpu.async_copy(idx_hbm.at[w * nck + i], idx_db.at[lane], sem)

    def idx_wait(i, lane, sem):
        pltpu.make_async_copy(
            idx_hbm.at[w * nck + i], idx_db.at[lane], sem).wait()

    def f_issue(i, flat_v, sem):
        pltpu.async_copy(
            vals_hbm.at[pl.ds((w * nck + i) * _CH * _EAW, _CH * _EAW)],
            flat_v, sem)

    def f_wait(i, flat_v, sem):
        pltpu.make_async_copy(
            vals_hbm.at[pl.ds((w * nck + i) * _CH * _EAW, _CH * _EAW)],
            flat_v, sem).wait()

    idx_issue(0, 0, sem_ia)
    idx_issue(1, 1, sem_ib)
    f_issue(0, flat_a, sem_fa)
    f_issue(1, flat_b, sem_fb)

    def step(i, lane, flat_v, sem_i, sem_f):
        idx_wait(i, lane, sem_i)
        f_wait(i, flat_v, sem_f)
        for k in range(_CH):
            rows_a[k, pl.ds(0, _EAW)] = flat_v[pl.ds(k * _EAW, _EAW)]
        pltpu.sync_copy(rows_a, acc_sh.at[idx_db.at[lane, 1]], add=True)

        @pl.when(i + 2 < nck)
        def _():
            idx_issue(i + 2, lane, sem_i)
            f_issue(i + 2, flat_v, sem_f)

    def pair(g, carry):
        step(2 * g, 0, flat_a, sem_ia, sem_fa)
        step(2 * g + 1, 1, flat_b, sem_ib, sem_fb)
        return carry

    lax.fori_loop(0, nck // 2, pair, 0)
    plsc.subcore_barrier()
    _copy_out(acc_sh, rows_a, out_hbm, r0, obase)


def _make_sc(gather, width):
    mesh = plsc.VectorSubcoreMesh(
        core_axis_name="c", subcore_axis_name="s",
        num_cores=_NC, num_subcores=_NS)
    if gather:
        body = _sc_gather_body
        scratch = [
            pltpu.VMEM_SHARED((_R, width), jnp.float32),
            pltpu.VMEM((2, 2, _CH), jnp.int32),
            pltpu.VMEM((_CH, width), jnp.float32),
            pltpu.VMEM((_CH, width), jnp.float32),
            pltpu.SemaphoreType.DMA,
            pltpu.SemaphoreType.DMA,
            pltpu.SemaphoreType.DMA,
            pltpu.SemaphoreType.DMA,
        ]
    else:
        body = _sc_ea_body
        scratch = [
            pltpu.VMEM_SHARED((_R, width), jnp.float32),
            pltpu.VMEM((2, 2, _CH), jnp.int32),
            pltpu.VMEM((_CH, width), jnp.float32),
            pltpu.VMEM((_CH * _EAW,), jnp.float32),
            pltpu.VMEM((_CH * _EAW,), jnp.float32),
            pltpu.SemaphoreType.DMA,
            pltpu.SemaphoreType.DMA,
            pltpu.SemaphoreType.DMA,
            pltpu.SemaphoreType.DMA,
        ]
    out_type = jax.ShapeDtypeStruct((_NC * _R, width), jnp.float32)
    return pl.kernel(
        body,
        out_type=out_type,
        mesh=mesh,
        scratch_types=scratch,
    )


_sc_seg_h = _make_sc(True, _EMB)
_sc_seg_ea = _make_sc(False, _EMB)


def _h0_body(x_ref, w_ref, b_ref, o_ref):
    o_ref[...] = (jnp.dot(x_ref[...], w_ref[...],
                          preferred_element_type=jnp.float32) + b_ref[...])


_tc_h0 = pl.pallas_call(
    _h0_body,
    grid=(_N // _BLK,),
    in_specs=[
        pl.BlockSpec((_BLK, 8), lambda i: (i, 0)),
        pl.BlockSpec((8, _EMB), lambda i: (0, 0)),
        pl.BlockSpec((1, _EMB), lambda i: (0, 0)),
    ],
    out_specs=pl.BlockSpec((_BLK, _EMB), lambda i: (i, 0)),
    out_shape=jax.ShapeDtypeStruct((_N, _EMB), jnp.float32),
)


def _mlp_body(p_ref, h_ref, a_ref, we_ref, c_ref, w1_ref, b1_ref, w2_ref,
              b2_ref, g_ref, s_ref, ss_ref):
    i = pl.program_id(0)
    agg = a_ref[0] + a_ref[1]
    aggr = (p_ref[0] + p_ref[1] + h_ref[...] + c_ref[...]
            + jnp.dot(agg, we_ref[...], preferred_element_type=jnp.float32))
    hid = jnp.maximum(
        jnp.dot(aggr, w1_ref[...], preferred_element_type=jnp.float32)
        + b1_ref[...], 0.0)
    g = (jnp.dot(hid, w2_ref[...], preferred_element_type=jnp.float32)
         + b2_ref[...])
    g_ref[...] = g

    @pl.when(i == 0)
    def _():
        s_ref[...] = jnp.zeros_like(s_ref)
        ss_ref[...] = jnp.zeros_like(ss_ref)

    s_ref[...] += jnp.sum(g, axis=0, keepdims=True)
    ss_ref[...] += jnp.sum(g * g, axis=0, keepdims=True)


_tc_mlp = pl.pallas_call(
    _mlp_body,
    grid=(_N // _BLK,),
    in_specs=[
        pl.BlockSpec((2, _BLK, _EMB), lambda i: (0, i, 0)),
        pl.BlockSpec((_BLK, _EMB), lambda i: (i, 0)),
        pl.BlockSpec((2, _BLK, _EMB), lambda i: (0, i, 0)),
        pl.BlockSpec((_EMB, _EMB), lambda i: (0, 0)),
        pl.BlockSpec((1, _EMB), lambda i: (0, 0)),
        pl.BlockSpec((_EMB, 2 * _EMB), lambda i: (0, 0)),
        pl.BlockSpec((1, 2 * _EMB), lambda i: (0, 0)),
        pl.BlockSpec((2 * _EMB, _EMB), lambda i: (0, 0)),
        pl.BlockSpec((1, _EMB), lambda i: (0, 0)),
    ],
    out_specs=[
        pl.BlockSpec((_BLK, _EMB), lambda i: (i, 0)),
        pl.BlockSpec((1, _EMB), lambda i: (0, 0)),
        pl.BlockSpec((1, _EMB), lambda i: (0, 0)),
    ],
    out_shape=[
        jax.ShapeDtypeStruct((_N, _EMB), jnp.float32),
        jax.ShapeDtypeStruct((1, _EMB), jnp.float32),
        jax.ShapeDtypeStruct((1, _EMB), jnp.float32),
    ],
)


def _bn_body(do_elu, g_ref, s_ref, ss_ref, gam_ref, bet_ref, o_ref):
    m = s_ref[...] * (1.0 / _N)
    v = ss_ref[...] * (1.0 / _N) - m * m
    y = gam_ref[...] * (g_ref[...] - m) * lax.rsqrt(v + 1e-5) + bet_ref[...]
    if do_elu:
        y = jnp.where(y > 0, y, jnp.exp(y) - 1.0)
    o_ref[...] = y


def _make_bn(do_elu):
    return pl.pallas_call(
        functools.partial(_bn_body, do_elu),
        grid=(_N // _BLK,),
        in_specs=[
            pl.BlockSpec((_BLK, _EMB), lambda i: (i, 0)),
            pl.BlockSpec((1, _EMB), lambda i: (0, 0)),
            pl.BlockSpec((1, _EMB), lambda i: (0, 0)),
            pl.BlockSpec((1, _EMB), lambda i: (0, 0)),
            pl.BlockSpec((1, _EMB), lambda i: (0, 0)),
        ],
        out_specs=pl.BlockSpec((_BLK, _EMB), lambda i: (i, 0)),
        out_shape=jax.ShapeDtypeStruct((_N, _EMB), jnp.float32),
    )


_tc_bn_elu = _make_bn(True)
_tc_bn = _make_bn(False)


def kernel(x, edge_index, edge_attr, Wx, bx, We0, be0, W1_0, b1_0, W2_0,
           b2_0, gamma0, beta0, We1, be1, W1_1, b1_1, W2_1, b2_1, gamma1,
           beta1):
    f32 = jnp.float32
    pad = _EP - _E

    srcp = jnp.concatenate([edge_index[0], jnp.zeros((pad,), jnp.int32)])
    dstp = jnp.concatenate(
        [edge_index[1], jnp.full((pad,), _DUMMY, jnp.int32)])
    ea16 = jnp.concatenate(
        [edge_attr, jnp.ones((_E, 1), f32), jnp.zeros((_E, _EAW - 6), f32)],
        axis=1)
    ea16 = jnp.concatenate([ea16, jnp.zeros((pad, _EAW), f32)], axis=0)
    zr = jnp.zeros((_CH, _EMB), f32)

    x8 = jnp.concatenate([x, jnp.zeros((_N, 1), f32)], axis=1)
    Wx8 = jnp.concatenate([Wx, jnp.zeros((1, _EMB), f32)], axis=0)
    h0 = _tc_h0(x8, Wx8, bx.reshape(1, -1))

    ei3 = jnp.stack([srcp.reshape(_EP // _CH, _CH),
                     dstp.reshape(_EP // _CH, _CH)], axis=1)
    P = _sc_seg_h(h0, ei3, zr)
    A = _sc_seg_ea(ea16.reshape(_EP * _EAW), ei3, zr)
    Pn = P.reshape(_NC, _R, _EMB)[:, :_N, :]
    An = A.reshape(_NC, _R, _EMB)[:, :_N, :]

    def dense_layer(h, Pn, An, We, be, W1, b1, W2, b2, gamma, beta, bn):
        Wem = jnp.concatenate(
            [We, be.reshape(1, -1), jnp.zeros((_EMB - 6, _EMB), f32)], axis=0)
        c = (8.0 * We[0] + be).reshape(1, -1)
        g, s, ss = _tc_mlp(Pn, h, An, Wem, c, W1, b1.reshape(1, -1), W2,
                           b2.reshape(1, -1))
        return bn(g, s, ss, gamma.reshape(1, -1), beta.reshape(1, -1))

    h1 = dense_layer(h0, Pn, An, We0, be0, W1_0, b1_0, W2_0, b2_0,
                     gamma0, beta0, _tc_bn_elu)

    P1 = _sc_seg_h(h1, ei3, zr).reshape(_NC, _R, _EMB)
    h2 = dense_layer(h1, P1[:, :_N, :], An, We1, be1, W1_1, b1_1, W2_1,
                     b2_1, gamma1, beta1, _tc_bn)
    return h2


# speedup vs baseline: 3.6524x; 1.0578x over previous
import functools

import jax
import jax.numpy as jnp
from jax import lax
from jax.experimental import pallas as pl
from jax.experimental.pallas import tpu as pltpu
from jax.experimental.pallas import tpu_sc as plsc

_N = 10000
_E = 320000
_EMB = 128

_NC = 2
_NS = 16
_NW = _NC * _NS
_CH = 128
_EW = 10240
_EP = _NW * _EW
_R = 10240
_RT = _R // _NS
_DUMMY = _N
_EAW = 16

_BLK = 1000

_CT = 160
_C0 = 122


def _sc_common(refs_sid):
    cid = lax.axis_index("c")
    sid = lax.axis_index("s")
    w = cid * _NS + sid
    r0 = sid * _RT
    obase = cid * _R + r0
    return cid, sid, w, r0, obase


def _zero_acc(z_hbm, rows_v, acc_sh, r0):
    pltpu.sync_copy(z_hbm, rows_v)
    for j in range(_RT // _CH):
        pltpu.sync_copy(rows_v, acc_sh.at[pl.ds(r0 + j * _CH, _CH)])


def _copy_out(acc_sh, rows_v, out_hbm, r0, obase):
    for j in range(_RT // _CH):
        pltpu.sync_copy(acc_sh.at[pl.ds(r0 + j * _CH, _CH)], rows_v)
        pltpu.sync_copy(rows_v, out_hbm.at[pl.ds(obase + j * _CH, _CH)])


def _sc_gather_body(vals_hbm, idx_hbm, z_hbm, out_hbm, acc_sh, idx_db,
                    rows_a, rows_b, sem_ia, sem_ib, sem_ga, sem_gb):
    cid, sid, w, r0, obase = _sc_common(None)

    cnt = jnp.where(cid == 0, _C0, _CT - _C0)
    cbase = jnp.where(cid == 0, sid * _C0,
                      _NS * _C0 + sid * (_CT - _C0))

    _zero_acc(z_hbm, rows_a, acc_sh, r0)
    plsc.subcore_barrier()

    def idx_issue(i, lane, sem):
        pltpu.async_copy(idx_hbm.at[cbase + i], idx_db.at[lane], sem)

    def idx_wait(i, lane, sem):
        pltpu.make_async_copy(
            idx_hbm.at[cbase + i], idx_db.at[lane], sem).wait()

    def g_issue(lane, rows_v, sem):
        pltpu.async_copy(vals_hbm.at[idx_db.at[lane, 0]], rows_v, sem)

    def g_wait(lane, rows_v, sem):
        pltpu.make_async_copy(
            vals_hbm.at[idx_db.at[lane, 0]], rows_v, sem).wait()

    idx_issue(0, 0, sem_ia)
    idx_issue(1, 1, sem_ib)
    idx_wait(0, 0, sem_ia)
    g_issue(0, rows_a, sem_ga)

    def pair(g, carry):
        i0 = 2 * g
        i1 = i0 + 1
        idx_wait(i1, 1, sem_ib)
        g_wait(0, rows_a, sem_ga)
        g_issue(1, rows_b, sem_gb)
        pltpu.sync_copy(rows_a, acc_sh.at[idx_db.at[0, 1]], add=True)

        @pl.when(i0 + 2 < cnt)
        def _():
            idx_issue(i0 + 2, 0, sem_ia)

        g_wait(1, rows_b, sem_gb)

        @pl.when(i0 + 2 < cnt)
        def _():
            idx_wait(i0 + 2, 0, sem_ia)
            g_issue(0, rows_a, sem_ga)

        pltpu.sync_copy(rows_b, acc_sh.at[idx_db.at[1, 1]], add=True)

        @pl.when(i1 + 2 < cnt)
        def _():
            idx_issue(i1 + 2, 1, sem_ib)

        return carry

    lax.fori_loop(0, cnt // 2, pair, 0)
    plsc.subcore_barrier()
    _copy_out(acc_sh, rows_a, out_hbm, r0, obase)


def _sc_ea_body(vals_hbm, idx_hbm, z_hbm, out_hbm, acc_sh, idx_db,
                rows_a, flat_a, flat_b, sem_ia, sem_ib, sem_fa, sem_fb):
    _, sid, w, r0, obase = _sc_common(None)
    nck = _EW // _CH

    _zero_acc(z_hbm, rows_a, acc_sh, r0)
    plsc.subcore_barrier()

    def idx_issue(i, lane, sem):
        pltpu.async_copy(idx_hbm.at[w * nck + i], idx_db.at[lane], sem)

    def idx_wait(i, lane, sem):
        pltpu.make_async_copy(
            idx_hbm.at[w * nck + i], idx_db.at[lane], sem).wait()

    def f_issue(i, flat_v, sem):
        pltpu.async_copy(
            vals_hbm.at[pl.ds((w * nck + i) * _CH * _EAW, _CH * _EAW)],
            flat_v, sem)

    def f_wait(i, flat_v, sem):
        pltpu.make_async_copy(
            vals_hbm.at[pl.ds((w * nck + i) * _CH * _EAW, _CH * _EAW)],
            flat_v, sem).wait()

    idx_issue(0, 0, sem_ia)
    idx_issue(1, 1, sem_ib)
    f_issue(0, flat_a, sem_fa)
    f_issue(1, flat_b, sem_fb)

    def step(i, lane, flat_v, sem_i, sem_f):
        idx_wait(i, lane, sem_i)
        f_wait(i, flat_v, sem_f)
        for k in range(_CH):
            rows_a[k, pl.ds(0, _EAW)] = flat_v[pl.ds(k * _EAW, _EAW)]
        pltpu.sync_copy(rows_a, acc_sh.at[idx_db.at[lane, 1]], add=True)

        @pl.when(i + 2 < nck)
        def _():
            idx_issue(i + 2, lane, sem_i)
            f_issue(i + 2, flat_v, sem_f)

    def pair(g, carry):
        step(2 * g, 0, flat_a, sem_ia, sem_fa)
        step(2 * g + 1, 1, flat_b, sem_ib, sem_fb)
        return carry

    lax.fori_loop(0, nck // 2, pair, 0)
    plsc.subcore_barrier()
    _copy_out(acc_sh, rows_a, out_hbm, r0, obase)


def _make_sc(gather, width):
    mesh = plsc.VectorSubcoreMesh(
        core_axis_name="c", subcore_axis_name="s",
        num_cores=_NC, num_subcores=_NS)
    if gather:
        body = _sc_gather_body
        scratch = [
            pltpu.VMEM_SHARED((_R, width), jnp.float32),
            pltpu.VMEM((2, 2, _CH), jnp.int32),
            pltpu.VMEM((_CH, width), jnp.float32),
            pltpu.VMEM((_CH, width), jnp.float32),
            pltpu.SemaphoreType.DMA,
            pltpu.SemaphoreType.DMA,
            pltpu.SemaphoreType.DMA,
            pltpu.SemaphoreType.DMA,
        ]
    else:
        body = _sc_ea_body
        scratch = [
            pltpu.VMEM_SHARED((_R, width), jnp.float32),
            pltpu.VMEM((2, 2, _CH), jnp.int32),
            pltpu.VMEM((_CH, width), jnp.float32),
            pltpu.VMEM((_CH * _EAW,), jnp.float32),
            pltpu.VMEM((_CH * _EAW,), jnp.float32),
            pltpu.SemaphoreType.DMA,
            pltpu.SemaphoreType.DMA,
            pltpu.SemaphoreType.DMA,
            pltpu.SemaphoreType.DMA,
        ]
    out_type = jax.ShapeDtypeStruct((_NC * _R, width), jnp.float32)
    return pl.kernel(
        body,
        out_type=out_type,
        mesh=mesh,
        scratch_types=scratch,
    )


_sc_seg_h = _make_sc(True, _EMB)
_sc_seg_ea = _make_sc(False, _EMB)


def _h0_body(x_ref, w_ref, b_ref, o_ref):
    o_ref[...] = (jnp.dot(x_ref[...], w_ref[...],
                          preferred_element_type=jnp.float32) + b_ref[...])


_tc_h0 = pl.pallas_call(
    _h0_body,
    grid=(_N // _BLK,),
    in_specs=[
        pl.BlockSpec((_BLK, 8), lambda i: (i, 0)),
        pl.BlockSpec((8, _EMB), lambda i: (0, 0)),
        pl.BlockSpec((1, _EMB), lambda i: (0, 0)),
    ],
    out_specs=pl.BlockSpec((_BLK, _EMB), lambda i: (i, 0)),
    out_shape=jax.ShapeDtypeStruct((_N, _EMB), jnp.float32),
)


def _mlp_body(p_ref, h_ref, a_ref, we_ref, c_ref, w1_ref, b1_ref, w2_ref,
              b2_ref, g_ref, s_ref, ss_ref):
    i = pl.program_id(0)
    agg = a_ref[0] + a_ref[1]
    aggr = (p_ref[0] + p_ref[1] + h_ref[...] + c_ref[...]
            + jnp.dot(agg, we_ref[...], preferred_element_type=jnp.float32))
    hid = jnp.maximum(
        jnp.dot(aggr, w1_ref[...], preferred_element_type=jnp.float32)
        + b1_ref[...], 0.0)
    g = (jnp.dot(hid, w2_ref[...], preferred_element_type=jnp.float32)
         + b2_ref[...])
    g_ref[...] = g

    @pl.when(i == 0)
    def _():
        s_ref[...] = jnp.zeros_like(s_ref)
        ss_ref[...] = jnp.zeros_like(ss_ref)

    s_ref[...] += jnp.sum(g, axis=0, keepdims=True)
    ss_ref[...] += jnp.sum(g * g, axis=0, keepdims=True)


_tc_mlp = pl.pallas_call(
    _mlp_body,
    grid=(_N // _BLK,),
    in_specs=[
        pl.BlockSpec((2, _BLK, _EMB), lambda i: (0, i, 0)),
        pl.BlockSpec((_BLK, _EMB), lambda i: (i, 0)),
        pl.BlockSpec((2, _BLK, _EMB), lambda i: (0, i, 0)),
        pl.BlockSpec((_EMB, _EMB), lambda i: (0, 0)),
        pl.BlockSpec((1, _EMB), lambda i: (0, 0)),
        pl.BlockSpec((_EMB, 2 * _EMB), lambda i: (0, 0)),
        pl.BlockSpec((1, 2 * _EMB), lambda i: (0, 0)),
        pl.BlockSpec((2 * _EMB, _EMB), lambda i: (0, 0)),
        pl.BlockSpec((1, _EMB), lambda i: (0, 0)),
    ],
    out_specs=[
        pl.BlockSpec((_BLK, _EMB), lambda i: (i, 0)),
        pl.BlockSpec((1, _EMB), lambda i: (0, 0)),
        pl.BlockSpec((1, _EMB), lambda i: (0, 0)),
    ],
    out_shape=[
        jax.ShapeDtypeStruct((_N, _EMB), jnp.float32),
        jax.ShapeDtypeStruct((1, _EMB), jnp.float32),
        jax.ShapeDtypeStruct((1, _EMB), jnp.float32),
    ],
)


def _bn_body(do_elu, g_ref, s_ref, ss_ref, gam_ref, bet_ref, o_ref):
    m = s_ref[...] * (1.0 / _N)
    v = ss_ref[...] * (1.0 / _N) - m * m
    y = gam_ref[...] * (g_ref[...] - m) * lax.rsqrt(v + 1e-5) + bet_ref[...]
    if do_elu:
        y = jnp.where(y > 0, y, jnp.exp(y) - 1.0)
    o_ref[...] = y


def _make_bn(do_elu):
    return pl.pallas_call(
        functools.partial(_bn_body, do_elu),
        grid=(_N // _BLK,),
        in_specs=[
            pl.BlockSpec((_BLK, _EMB), lambda i: (i, 0)),
            pl.BlockSpec((1, _EMB), lambda i: (0, 0)),
            pl.BlockSpec((1, _EMB), lambda i: (0, 0)),
            pl.BlockSpec((1, _EMB), lambda i: (0, 0)),
            pl.BlockSpec((1, _EMB), lambda i: (0, 0)),
        ],
        out_specs=pl.BlockSpec((_BLK, _EMB), lambda i: (i, 0)),
        out_shape=jax.ShapeDtypeStruct((_N, _EMB), jnp.float32),
    )


_tc_bn_elu = _make_bn(True)
_tc_bn = _make_bn(False)


def kernel(x, edge_index, edge_attr, Wx, bx, We0, be0, W1_0, b1_0, W2_0,
           b2_0, gamma0, beta0, We1, be1, W1_1, b1_1, W2_1, b2_1, gamma1,
           beta1):
    f32 = jnp.float32
    pad = _EP - _E

    srcp = jnp.concatenate([edge_index[0], jnp.zeros((pad,), jnp.int32)])
    dstp = jnp.concatenate(
        [edge_index[1], jnp.full((pad,), _DUMMY, jnp.int32)])
    ea16 = jnp.concatenate(
        [edge_attr, jnp.ones((_E, 1), f32), jnp.zeros((_E, _EAW - 6), f32)],
        axis=1)
    ea16 = jnp.concatenate([ea16, jnp.zeros((pad, _EAW), f32)], axis=0)
    zr = jnp.zeros((_CH, _EMB), f32)

    x8 = jnp.concatenate([x, jnp.zeros((_N, 1), f32)], axis=1)
    Wx8 = jnp.concatenate([Wx, jnp.zeros((1, _EMB), f32)], axis=0)
    h0 = _tc_h0(x8, Wx8, bx.reshape(1, -1))

    ei3 = jnp.stack([srcp.reshape(_EP // _CH, _CH),
                     dstp.reshape(_EP // _CH, _CH)], axis=1)
    P = _sc_seg_h(h0, ei3, zr)
    A = _sc_seg_ea(ea16.reshape(_EP * _EAW), ei3, zr)
    Pn = P.reshape(_NC, _R, _EMB)[:, :_N, :]
    An = A.reshape(_NC, _R, _EMB)[:, :_N, :]

    def dense_layer(h, Pn, An, We, be, W1, b1, W2, b2, gamma, beta, bn):
        Wem = jnp.concatenate(
            [We, be.reshape(1, -1), jnp.zeros((_EMB - 6, _EMB), f32)], axis=0)
        c = (8.0 * We[0] + be).reshape(1, -1)
        g, s, ss = _tc_mlp(Pn, h, An, Wem, c, W1, b1.reshape(1, -1), W2,
                           b2.reshape(1, -1))
        return bn(g, s, ss, gamma.reshape(1, -1), beta.reshape(1, -1))

    h1 = dense_layer(h0, Pn, An, We0, be0, W1_0, b1_0, W2_0, b2_0,
                     gamma0, beta0, _tc_bn_elu)

    P1 = _sc_seg_h(h1, ei3, zr).reshape(_NC, _R, _EMB)
    h2 = dense_layer(h1, P1[:, :_N, :], An, We1, be1, W1_1, b1_1, W2_1,
                     b2_1, gamma1, beta1, _tc_bn)
    return h2
